# baseline (speedup 1.0000x reference)
import functools

import jax
import jax.numpy as jnp
from jax import lax
from jax.experimental import pallas as pl
from jax.experimental.pallas import tpu as pltpu
from jax.experimental.pallas import tpu_sc as plsc


_CHUNK = 2048
_NW = 32
_IDXW = 128
_SUPER = 1024
_K = 128


def _sc_gather_add_relu(u, v, src2d, dst2d, e_pad):
    n, d = u.shape
    bf = d == 256
    if bf:
        u = jax.lax.bitcast_convert_type(
            u.astype(jnp.bfloat16).reshape(n, d // 2, 2), jnp.float32)
        v = jax.lax.bitcast_convert_type(
            v.astype(jnp.bfloat16).reshape(n, d // 2, 2), jnp.float32)
    dw = u.shape[1]
    buf_t = pltpu.VMEM((2, _K, dw), jnp.float32)
    if bf:
        out_t = (jax.ShapeDtypeStruct((e_pad, dw), jnp.float32),
                 jax.ShapeDtypeStruct((e_pad, dw), jnp.float32))
    else:
        out_t = jax.ShapeDtypeStruct((e_pad, dw), jnp.float32)
    per_w = e_pad // _NW
    t_total = per_w // _K
    nsub = _SUPER // _K
    assert per_w % _SUPER == 0 and e_pad % (_NW * _SUPER) == 0 and t_total >= 4

    mesh = plsc.VectorSubcoreMesh(core_axis_name="c", subcore_axis_name="s")

    @functools.partial(
        pl.kernel,
        out_type=out_t,
        mesh=mesh,
        scratch_types=[
            pltpu.VMEM((2, nsub, _IDXW), jnp.int32),
            pltpu.VMEM((2, nsub, _IDXW), jnp.int32),
            buf_t,
            buf_t,
            pltpu.SemaphoreType.DMA((2,)),
            pltpu.SemaphoreType.DMA((2,)),
            pltpu.SemaphoreType.DMA((2,)),
        ],
    )
    def gather_kernel(u_hbm, v_hbm, src_hbm, dst_hbm, *rest):
        if bf:
            (out_hbm, outv_hbm, si, di, bu, bv, su, sv, so) = rest
        else:
            (out_hbm, si, di, bu, bv, su, sv, so) = rest
            outv_hbm = None
        wid = lax.axis_index("s") * 2 + lax.axis_index("c")
        row_base = wid * per_w

        def _compute(b1):
            def row_body(r, c2):
                for c0 in range(dw // 16):
                    sl = pl.ds(c0 * 16, 16)
                    bu[b1, r, sl] = jnp.maximum(bu[b1, r, sl] + bv[b1, r, sl],
                                                0.0)
                return c2

            lax.fori_loop(0, _K, row_body, 0)

        def _wait_gathers(b1):
            pltpu.make_async_copy(u_hbm.at[si.at[0, 0]], bu.at[b1],
                                  su.at[b1]).wait()
            pltpu.make_async_copy(v_hbm.at[di.at[0, 0]], bv.at[b1],
                                  sv.at[b1]).wait()

        def _wait_out(b1):
            pltpu.make_async_copy(bu.at[b1], out_hbm.at[pl.ds(0, _K)],
                                  so.at[b1]).wait()
            if bf:
                pltpu.make_async_copy(bv.at[b1], outv_hbm.at[pl.ds(0, _K)],
                                      so.at[b1]).wait()

        def _fire_out(b1, rows):
            pltpu.async_copy(bu.at[b1], out_hbm.at[pl.ds(rows, _K)],
                             so.at[b1])
            if bf:
                pltpu.async_copy(bv.at[b1], outv_hbm.at[pl.ds(rows, _K)],
                                 so.at[b1])

        def body_t(t, carry):
            b = lax.rem(t, 2)
            sb = lax.rem(t // nsub, 2)
            r = lax.rem(t, nsub)

            @pl.when(r == 0)
            def _stage():
                ioff = pl.multiple_of((row_base + t * _K) // _IDXW, nsub)
                pltpu.sync_copy(src_hbm.at[pl.ds(ioff, nsub)], si.at[sb])
                pltpu.sync_copy(dst_hbm.at[pl.ds(ioff, nsub)], di.at[sb])

            @pl.when(t >= 2)
            def _wo():
                _wait_out(b)

            pltpu.async_copy(u_hbm.at[si.at[sb, r]], bu.at[b], su.at[b])
            pltpu.async_copy(v_hbm.at[di.at[sb, r]], bv.at[b], sv.at[b])

            @pl.when(t >= 1)
            def _finish_prev():
                b1 = lax.rem(t + 1, 2)
                _wait_gathers(b1)
                if not bf:
                    _compute(b1)
                rows = pl.multiple_of(row_base + (t - 1) * _K, _K)
                _fire_out(b1, rows)

            return carry

        lax.fori_loop(0, t_total, body_t, 0)

        b_last = (t_total - 1) % 2
        _wait_gathers(b_last)
        if not bf:
            _compute(b_last)
        rows_last = pl.multiple_of(row_base + (t_total - 1) * _K, _K)
        pltpu.sync_copy(bu.at[b_last], out_hbm.at[pl.ds(rows_last, _K)])
        if bf:
            pltpu.sync_copy(bv.at[b_last], outv_hbm.at[pl.ds(rows_last, _K)])
        _wait_out(t_total % 2)

    out = gather_kernel(u, v, src2d, dst2d)
    if bf:
        gu = jax.lax.bitcast_convert_type(out[0],
                                          jnp.bfloat16).reshape(e_pad, d)
        gv = jax.lax.bitcast_convert_type(out[1],
                                          jnp.bfloat16).reshape(e_pad, d)
        return (gu, gv)
    return out


def _edge_mlp_body(h1_ref, w2_ref, b2_ref, w3_ref, b3_ref, out_ref):
    h1 = h1_ref[...].astype(jnp.bfloat16)
    h2 = jnp.dot(h1, w2_ref[...], preferred_element_type=jnp.float32)
    h2 = jnp.maximum(h2 + b2_ref[...], 0.0)
    out_ref[...] = (
        jnp.dot(h2.astype(jnp.bfloat16), w3_ref[...],
                preferred_element_type=jnp.float32) + b3_ref[...]
    )


def _edge_mlp(h1, w2, b2, w3, b3):
    e_pad, d1 = h1.shape
    d2 = w2.shape[1]
    d3 = w3.shape[1]
    grid = e_pad // _CHUNK
    return pl.pallas_call(
        _edge_mlp_body,
        grid=(grid,),
        in_specs=[
            pl.BlockSpec((_CHUNK, d1), lambda i: (i, 0)),
            pl.BlockSpec((d1, d2), lambda i: (0, 0)),
            pl.BlockSpec((1, d2), lambda i: (0, 0)),
            pl.BlockSpec((d2, d3), lambda i: (0, 0)),
            pl.BlockSpec((1, d3), lambda i: (0, 0)),
        ],
        out_specs=pl.BlockSpec((_CHUNK, d3), lambda i: (i, 0)),
        out_shape=jax.ShapeDtypeStruct((e_pad, d3), jnp.float32),
    )(h1, w2.astype(jnp.bfloat16), b2.reshape(1, -1), w3.astype(jnp.bfloat16),
      b3.reshape(1, -1))


def _edge_mlp_graphmax_body(pair, *refs):
    if pair:
        (gu_ref, gv_ref, pen_ref, w2_ref, b2_ref, w3_ref, b3_ref,
         out_ref) = refs
        h1 = jnp.maximum(gu_ref[...] + gv_ref[...], jnp.bfloat16(0))
    else:
        h1_ref, pen_ref, w2_ref, b2_ref, w3_ref, b3_ref, out_ref = refs
        h1 = h1_ref[...].astype(jnp.bfloat16)

    @pl.when(pl.program_id(0) == 0)
    def _init():
        out_ref[...] = jnp.full_like(out_ref, -jnp.inf)

    h2 = jnp.dot(h1, w2_ref[...], preferred_element_type=jnp.float32)
    h2 = jnp.maximum(h2 + b2_ref[...], 0.0)
    h3 = (jnp.dot(h2.astype(jnp.bfloat16), w3_ref[...],
                  preferred_element_type=jnp.float32) + b3_ref[...])
    pen = pen_ref[...]
    rows = [
        jnp.max(h3 + pen[:, g:g + 1], axis=0, keepdims=True) for g in range(8)
    ]
    out_ref[...] = jnp.maximum(out_ref[...], jnp.concatenate(rows, axis=0))


def _edge_mlp_graphmax(h1, pen, w2, b2, w3, b3):
    pair = isinstance(h1, tuple)
    arrs = h1 if pair else (h1,)
    e_pad, d1 = arrs[0].shape
    d2 = w2.shape[1]
    d3 = w3.shape[1]
    grid = e_pad // _CHUNK
    return pl.pallas_call(
        functools.partial(_edge_mlp_graphmax_body, pair),
        grid=(grid,),
        in_specs=[pl.BlockSpec((_CHUNK, d1), lambda i: (i, 0))
                  for _ in arrs] + [
            pl.BlockSpec((_CHUNK, 8), lambda i: (i, 0)),
            pl.BlockSpec((d1, d2), lambda i: (0, 0)),
            pl.BlockSpec((1, d2), lambda i: (0, 0)),
            pl.BlockSpec((d2, d3), lambda i: (0, 0)),
            pl.BlockSpec((1, d3), lambda i: (0, 0)),
        ],
        out_specs=pl.BlockSpec((8, d3), lambda i: (0, 0)),
        out_shape=jax.ShapeDtypeStruct((8, d3), jnp.float32),
    )(*arrs, pen, w2.astype(jnp.bfloat16), b2.reshape(1, -1),
      w3.astype(jnp.bfloat16), b3.reshape(1, -1))


def _conv_h1(layers, x, pos, src, dst, n):
    (w1, b1), (w2, b2), (w3, b3) = layers
    c = x.shape[1]
    w1x, w1p = w1[:c], w1[c:]
    u = x @ w1x + pos @ w1p + b1
    v = -(pos @ w1p)

    d1 = u.shape[1]
    if d1 % 128:
        d1p = ((d1 + 127) // 128) * 128
        u = jnp.pad(u, ((0, 0), (0, d1p - d1)))
        v = jnp.pad(v, ((0, 0), (0, d1p - d1)))
        w2 = jnp.pad(w2, ((0, d1p - d1), (0, 0)))

    e = src.shape[0]
    align = 2 * _NW * _SUPER
    e_pad = ((e + align - 1) // align) * align
    pad = e_pad - e
    spread = (jnp.arange(e_pad, dtype=src.dtype) * 7) % n
    gsrc = jnp.where(jnp.arange(e_pad) < e, jnp.pad(src, (0, pad)), spread)
    gdst = jnp.where(jnp.arange(e_pad) < e, jnp.pad(dst, (0, pad)), spread)
    seg = jnp.pad(dst, (0, pad), constant_values=n)

    half = e_pad // 2
    halves = []
    for i in range(2):
        sl = slice(i * half, (i + 1) * half)
        h1 = _sc_gather_add_relu(u, v, gsrc[sl].reshape(-1, _IDXW),
                                 gdst[sl].reshape(-1, _IDXW), half)
        halves.append((h1, seg[sl]))
    return halves, w2


def _point_conv_opt(layers, x, pos, src, dst, n):
    halves, w2 = _conv_h1(layers, x, pos, src, dst, n)
    (_, _), (_, b2), (w3, b3) = layers
    aggs = []
    for h1, seg in halves:
        h3 = _edge_mlp(h1, w2, b2, w3, b3)
        aggs.append(jax.ops.segment_max(h3, seg, num_segments=n + 1)[:n])
    agg = jnp.maximum(aggs[0], aggs[1])
    return jnp.where(jnp.isfinite(agg), agg, 0.0)


def _point_conv_graphmax(layers, x, pos, src, dst, n, node_graph, ng):
    halves, w2 = _conv_h1(layers, x, pos, src, dst, n)
    (_, _), (_, b2), (w3, b3) = layers
    starts = jnp.searchsorted(node_graph, jnp.arange(1, ng, dtype=node_graph.dtype))
    gs = []
    for h1, seg in halves:
        gid = jnp.zeros(seg.shape, jnp.int32)
        for b in range(ng - 1):
            gid = gid + (seg >= starts[b]).astype(jnp.int32)
        gid = jnp.where(seg >= n, ng, gid)
        pen = jnp.where(gid[:, None] == jnp.arange(ng)[None, :], 0.0, -jnp.inf)
        gs.append(_edge_mlp_graphmax(h1, pen.astype(jnp.float32), w2, b2, w3,
                                     b3))
    g = jnp.maximum(gs[0], gs[1])
    return jnp.where(jnp.isfinite(g), g, 0.0)


def _pool_opt(x, pos, edge_index, batch, add_self_loops):
    n = x.shape[0]
    n2 = n // 2
    xp = x[: n2 * 2].reshape(n2, 2, -1).mean(axis=1)
    pp = pos[: n2 * 2].reshape(n2, 2, -1).mean(axis=1)
    ei = jnp.minimum(edge_index // 2, n2 - 1)
    if add_self_loops:
        loop = jnp.arange(n2, dtype=ei.dtype)
        ei = jnp.concatenate([ei, jnp.stack([loop, loop])], axis=1)
    bp = batch[: n2 * 2 : 2]
    return xp, pp, ei, bp


def kernel(x, pos, params, edge_index, batch):
    n = x.shape[0]
    loop = jnp.arange(n, dtype=edge_index.dtype)
    src1 = jnp.concatenate([edge_index[0], loop])
    dst1 = jnp.concatenate([edge_index[1], loop])

    x1 = _point_conv_opt(params["conv1"], x, pos, src1, dst1, n)
    x1, pos1, ei1, b1 = _pool_opt(x1, pos, edge_index, batch, True)
    x2 = _point_conv_opt(params["conv2"], x1, pos1, ei1[0], ei1[1], x1.shape[0])
    x2, pos2, ei2, b2 = _pool_opt(x2, pos1, ei1, b1, False)
    g = _point_conv_graphmax(params["conv3"], x2, pos2, ei2[0], ei2[1],
                             x2.shape[0], b2, 8)

    g1, bb1 = params["bn1"]
    h = jax.nn.relu(g1 * g + bb1)
    h = h @ params["lin1"][0] + params["lin1"][1]
    g2, bb2 = params["bn2"]
    h = jax.nn.relu(g2 * h + bb2)
    h = h @ params["lin2"][0] + params["lin2"][1]
    g3, bb3 = params["bn3"]
    h = jax.nn.relu(g3 * h + bb3)
    return h @ params["lin3"][0] + params["lin3"][1]


# speedup vs baseline: 1.7251x; 1.7251x over previous
import functools

import jax
import jax.numpy as jnp
from jax import lax
from jax.experimental import pallas as pl
from jax.experimental.pallas import tpu as pltpu
from jax.experimental.pallas import tpu_sc as plsc


_CHUNK = 2048
_NW = 32
_IDXW = 128
_SUPER = 1024


def _sc_gather_add_relu(u, v, src2d, dst2d, e_pad):
    n, d = u.shape
    k = 32768 // d
    kg = k // _IDXW
    sup = _SUPER // k
    per_w = e_pad // _NW
    n_super = per_w // _SUPER
    assert per_w % _SUPER == 0 and e_pad % (_NW * _SUPER) == 0

    mesh = plsc.VectorSubcoreMesh(core_axis_name="c", subcore_axis_name="s")

    @functools.partial(
        pl.kernel,
        out_type=jax.ShapeDtypeStruct((e_pad, d), jnp.float32),
        mesh=mesh,
        scratch_types=[
            pltpu.VMEM((_SUPER // _IDXW, _IDXW), jnp.int32),
            pltpu.VMEM((_SUPER // _IDXW, _IDXW), jnp.int32),
            pltpu.VMEM((k, d), jnp.float32),
            pltpu.VMEM((k, d), jnp.float32),
            pltpu.SemaphoreType.DMA,
            pltpu.SemaphoreType.DMA,
        ],
    )
    def gather_kernel(u_hbm, v_hbm, src_hbm, dst_hbm, out_hbm, si, di, bu, bv,
                      su, sv):
        wid = lax.axis_index("s") * 2 + lax.axis_index("c")
        row_base = wid * per_w

        def super_body(ci, carry):
            soff = pl.multiple_of(row_base + ci * _SUPER, _SUPER)
            ioff = pl.multiple_of(soff // _IDXW, _SUPER // _IDXW)
            pltpu.sync_copy(src_hbm.at[pl.ds(ioff, _SUPER // _IDXW)], si)
            pltpu.sync_copy(dst_hbm.at[pl.ds(ioff, _SUPER // _IDXW)], di)
            for s in range(sup):
                cus = [
                    pltpu.async_copy(u_hbm.at[si.at[s * kg + j]],
                                     bu.at[pl.ds(j * _IDXW, _IDXW)], su)
                    for j in range(kg)
                ]
                cvs = [
                    pltpu.async_copy(v_hbm.at[di.at[s * kg + j]],
                                     bv.at[pl.ds(j * _IDXW, _IDXW)], sv)
                    for j in range(kg)
                ]
                for c in cus:
                    c.wait()
                for c in cvs:
                    c.wait()

                def row_body(r, c2):
                    for c0 in range(d // 16):
                        sl = pl.ds(c0 * 16, 16)
                        bu[r, sl] = jnp.maximum(bu[r, sl] + bv[r, sl], 0.0)
                    return c2

                lax.fori_loop(0, k, row_body, 0)
                pltpu.sync_copy(
                    bu, out_hbm.at[pl.ds(pl.multiple_of(soff + s * k, k), k)])
            return carry

        lax.fori_loop(0, n_super, super_body, 0)

    return gather_kernel(u, v, src2d, dst2d)


def _edge_mlp_body(h1_ref, w2_ref, b2_ref, w3_ref, b3_ref, out_ref):
    h1 = h1_ref[...].astype(jnp.bfloat16)
    h2 = jnp.dot(h1, w2_ref[...], preferred_element_type=jnp.float32)
    h2 = jnp.maximum(h2 + b2_ref[...], 0.0)
    out_ref[...] = (
        jnp.dot(h2.astype(jnp.bfloat16), w3_ref[...],
                preferred_element_type=jnp.float32) + b3_ref[...]
    )


def _edge_mlp(h1, w2, b2, w3, b3):
    e_pad, d1 = h1.shape
    d2 = w2.shape[1]
    d3 = w3.shape[1]
    grid = e_pad // _CHUNK
    return pl.pallas_call(
        _edge_mlp_body,
        grid=(grid,),
        in_specs=[
            pl.BlockSpec((_CHUNK, d1), lambda i: (i, 0)),
            pl.BlockSpec((d1, d2), lambda i: (0, 0)),
            pl.BlockSpec((1, d2), lambda i: (0, 0)),
            pl.BlockSpec((d2, d3), lambda i: (0, 0)),
            pl.BlockSpec((1, d3), lambda i: (0, 0)),
        ],
        out_specs=pl.BlockSpec((_CHUNK, d3), lambda i: (i, 0)),
        out_shape=jax.ShapeDtypeStruct((e_pad, d3), jnp.float32),
    )(h1, w2.astype(jnp.bfloat16), b2.reshape(1, -1), w3.astype(jnp.bfloat16),
      b3.reshape(1, -1))


def _edge_mlp_graphmax_body(h1_ref, pen_ref, w2_ref, b2_ref, w3_ref, b3_ref,
                            out_ref):
    @pl.when(pl.program_id(0) == 0)
    def _init():
        out_ref[...] = jnp.full_like(out_ref, -jnp.inf)

    h1 = h1_ref[...].astype(jnp.bfloat16)
    h2 = jnp.dot(h1, w2_ref[...], preferred_element_type=jnp.float32)
    h2 = jnp.maximum(h2 + b2_ref[...], 0.0)
    h3 = (jnp.dot(h2.astype(jnp.bfloat16), w3_ref[...],
                  preferred_element_type=jnp.float32) + b3_ref[...])
    pen = pen_ref[...]
    rows = [
        jnp.max(h3 + pen[:, g:g + 1], axis=0, keepdims=True) for g in range(8)
    ]
    out_ref[...] = jnp.maximum(out_ref[...], jnp.concatenate(rows, axis=0))


def _edge_mlp_graphmax(h1, pen, w2, b2, w3, b3):
    e_pad, d1 = h1.shape
    d2 = w2.shape[1]
    d3 = w3.shape[1]
    grid = e_pad // _CHUNK
    return pl.pallas_call(
        _edge_mlp_graphmax_body,
        grid=(grid,),
        in_specs=[
            pl.BlockSpec((_CHUNK, d1), lambda i: (i, 0)),
            pl.BlockSpec((_CHUNK, 8), lambda i: (i, 0)),
            pl.BlockSpec((d1, d2), lambda i: (0, 0)),
            pl.BlockSpec((1, d2), lambda i: (0, 0)),
            pl.BlockSpec((d2, d3), lambda i: (0, 0)),
            pl.BlockSpec((1, d3), lambda i: (0, 0)),
        ],
        out_specs=pl.BlockSpec((8, d3), lambda i: (0, 0)),
        out_shape=jax.ShapeDtypeStruct((8, d3), jnp.float32),
    )(h1, pen, w2.astype(jnp.bfloat16), b2.reshape(1, -1),
      w3.astype(jnp.bfloat16), b3.reshape(1, -1))


def _conv_h1(layers, x, pos, src, dst, n):
    (w1, b1), (w2, b2), (w3, b3) = layers
    c = x.shape[1]
    w1x, w1p = w1[:c], w1[c:]
    u = x @ w1x + pos @ w1p + b1
    v = -(pos @ w1p)

    d1 = u.shape[1]
    if d1 % 128:
        d1p = ((d1 + 127) // 128) * 128
        u = jnp.pad(u, ((0, 0), (0, d1p - d1)))
        v = jnp.pad(v, ((0, 0), (0, d1p - d1)))
        w2 = jnp.pad(w2, ((0, d1p - d1), (0, 0)))

    e = src.shape[0]
    align = 2 * _NW * _SUPER
    e_pad = ((e + align - 1) // align) * align
    pad = e_pad - e
    spread = (jnp.arange(e_pad, dtype=src.dtype) * 7) % n
    gsrc = jnp.where(jnp.arange(e_pad) < e, jnp.pad(src, (0, pad)), spread)
    gdst = jnp.where(jnp.arange(e_pad) < e, jnp.pad(dst, (0, pad)), spread)
    seg = jnp.pad(dst, (0, pad), constant_values=n)

    half = e_pad // 2
    halves = []
    for i in range(2):
        sl = slice(i * half, (i + 1) * half)
        h1 = _sc_gather_add_relu(u, v, gsrc[sl].reshape(-1, _IDXW),
                                 gdst[sl].reshape(-1, _IDXW), half)
        halves.append((h1, seg[sl]))
    return halves, w2


def _point_conv_opt(layers, x, pos, src, dst, n):
    halves, w2 = _conv_h1(layers, x, pos, src, dst, n)
    (_, _), (_, b2), (w3, b3) = layers
    aggs = []
    for h1, seg in halves:
        h3 = _edge_mlp(h1, w2, b2, w3, b3)
        aggs.append(jax.ops.segment_max(h3, seg, num_segments=n + 1)[:n])
    agg = jnp.maximum(aggs[0], aggs[1])
    return jnp.where(jnp.isfinite(agg), agg, 0.0)


def _point_conv_graphmax(layers, x, pos, src, dst, n, node_graph, ng):
    halves, w2 = _conv_h1(layers, x, pos, src, dst, n)
    (_, _), (_, b2), (w3, b3) = layers
    starts = jnp.searchsorted(node_graph, jnp.arange(1, ng, dtype=node_graph.dtype))
    gs = []
    for h1, seg in halves:
        gid = jnp.zeros(seg.shape, jnp.int32)
        for b in range(ng - 1):
            gid = gid + (seg >= starts[b]).astype(jnp.int32)
        gid = jnp.where(seg >= n, ng, gid)
        pen = jnp.where(gid[:, None] == jnp.arange(ng)[None, :], 0.0, -jnp.inf)
        gs.append(_edge_mlp_graphmax(h1, pen.astype(jnp.float32), w2, b2, w3,
                                     b3))
    g = jnp.maximum(gs[0], gs[1])
    return jnp.where(jnp.isfinite(g), g, 0.0)


def _pool_opt(x, pos, edge_index, batch, add_self_loops):
    n = x.shape[0]
    n2 = n // 2
    xp = x[: n2 * 2].reshape(n2, 2, -1).mean(axis=1)
    pp = pos[: n2 * 2].reshape(n2, 2, -1).mean(axis=1)
    ei = jnp.minimum(edge_index // 2, n2 - 1)
    if add_self_loops:
        loop = jnp.arange(n2, dtype=ei.dtype)
        ei = jnp.concatenate([ei, jnp.stack([loop, loop])], axis=1)
    bp = batch[: n2 * 2 : 2]
    return xp, pp, ei, bp


def kernel(x, pos, params, edge_index, batch):
    n = x.shape[0]
    loop = jnp.arange(n, dtype=edge_index.dtype)
    src1 = jnp.concatenate([edge_index[0], loop])
    dst1 = jnp.concatenate([edge_index[1], loop])

    x1 = _point_conv_opt(params["conv1"], x, pos, src1, dst1, n)
    x1, pos1, ei1, b1 = _pool_opt(x1, pos, edge_index, batch, True)
    x2 = _point_conv_opt(params["conv2"], x1, pos1, ei1[0], ei1[1], x1.shape[0])
    x2, pos2, ei2, b2 = _pool_opt(x2, pos1, ei1, b1, False)
    g = _point_conv_graphmax(params["conv3"], x2, pos2, ei2[0], ei2[1],
                             x2.shape[0], b2, 8)

    g1, bb1 = params["bn1"]
    h = jax.nn.relu(g1 * g + bb1)
    h = h @ params["lin1"][0] + params["lin1"][1]
    g2, bb2 = params["bn2"]
    h = jax.nn.relu(g2 * h + bb2)
    h = h @ params["lin2"][0] + params["lin2"][1]
    g3, bb3 = params["bn3"]
    h = jax.nn.relu(g3 * h + bb3)
    return h @ params["lin3"][0] + params["lin3"][1]
